# Initial kernel scaffold; baseline (speedup 1.0000x reference)
import jax
import jax.numpy as jnp
from jax.experimental import pallas as pl


def kernel(item_hist_seq, cate_hist_seq, user_seq, item_seq, s_num_feats, T_item_hist, T_cate_hist, T_user, T_item, W1, b1, W2, b2, W3, b3, Wc, bc):
    raise NotImplementedError("write your pallas kernel here")


# speedup vs baseline: 6.0158x; 6.0158x over previous
import functools

import jax
import jax.numpy as jnp
from jax import lax
from jax.experimental import pallas as pl
from jax.experimental.pallas import tpu as pltpu
from jax.experimental.pallas import tpu_sc as plsc

B = 4096
L = 50
D_ITEM = 64
D_CATE = 32
FF_IN = 128
HID = 256

NC = 2
NS = 16
NW = NC * NS
RPW = B // NW
GROUP = 2
GPW = RPW // GROUP
IPG = GROUP * L


def _sc_body(ihs, chs, us, its, Ti, Tc, Tu, Tt,
             sum1, sum2, eu, et,
             idx_i, idx_c, idx_u, idx_t,
             rows_i, rows_c, o1, o2, oeu, oet,
             sem_i, sem_c, sem_u, sem_t):
    c = lax.axis_index("c")
    s = lax.axis_index("s")
    w = s * NC + c
    base = w * RPW
    gbase = w * GPW

    pltpu.sync_copy(ihs.at[pl.ds(gbase, GPW)], idx_i)
    pltpu.sync_copy(chs.at[pl.ds(gbase, GPW)], idx_c)
    pltpu.sync_copy(us.at[pl.ds(base, RPW)], idx_u)
    pltpu.sync_copy(its.at[pl.ds(base, RPW)], idx_t)

    cp_u = pltpu.async_copy(Tu.at[idx_u], oeu, sem_u)
    cp_t = pltpu.async_copy(Tt.at[idx_t], oet, sem_t)

    pltpu.async_copy(Ti.at[idx_i.at[0]], rows_i.at[0], sem_i)
    pltpu.async_copy(Tc.at[idx_c.at[0]], rows_c.at[0], sem_c)

    def step(h, _):
        for bb in range(2):
            g = h * 2 + bb
            pltpu.make_async_copy(Ti.at[idx_i.at[g]], rows_i.at[bb], sem_i).wait()
            pltpu.make_async_copy(Tc.at[idx_c.at[g]], rows_c.at[bb], sem_c).wait()

            @pl.when(g + 1 < GPW)
            def _():
                pltpu.async_copy(Ti.at[idx_i.at[g + 1]], rows_i.at[1 - bb], sem_i)
                pltpu.async_copy(Tc.at[idx_c.at[g + 1]], rows_c.at[1 - bb], sem_c)

            for r in range(GROUP):
                lr = g * GROUP + r
                acc_i = [rows_i[bb, r * L, pl.ds(v * 16, 16)] for v in range(4)]
                acc_c = [rows_c[bb, r * L, pl.ds(v * 16, 16)] for v in range(2)]
                for j in range(1, L):
                    for v in range(4):
                        acc_i[v] = acc_i[v] + rows_i[bb, r * L + j, pl.ds(v * 16, 16)]
                    for v in range(2):
                        acc_c[v] = acc_c[v] + rows_c[bb, r * L + j, pl.ds(v * 16, 16)]
                for v in range(4):
                    o1[lr, pl.ds(v * 16, 16)] = acc_i[v]
                for v in range(2):
                    o2[lr, pl.ds(v * 16, 16)] = acc_c[v]
        return ()

    lax.fori_loop(0, GPW // 2, step, (), unroll=False)

    cp_u.wait()
    cp_t.wait()

    pltpu.sync_copy(o1, sum1.at[pl.ds(base, RPW)])
    pltpu.sync_copy(o2, sum2.at[pl.ds(base, RPW)])
    pltpu.sync_copy(oeu, eu.at[pl.ds(base, RPW)])
    pltpu.sync_copy(oet, et.at[pl.ds(base, RPW)])


def _sc_pool(ihs, chs, us, its, Ti, Tc, Tu, Tt):
    mesh = plsc.VectorSubcoreMesh(core_axis_name="c", subcore_axis_name="s",
                                  num_cores=NC, num_subcores=NS)
    f32 = jnp.float32
    kern = pl.kernel(
        _sc_body,
        out_type=(
            jax.ShapeDtypeStruct((B, D_ITEM), f32),
            jax.ShapeDtypeStruct((B, D_CATE), f32),
            jax.ShapeDtypeStruct((B, D_ITEM), f32),
            jax.ShapeDtypeStruct((B, D_ITEM), f32),
        ),
        mesh=mesh,
        compiler_params=pltpu.CompilerParams(use_tc_tiling_on_sc=False),
        scratch_types=[
            pltpu.VMEM((GPW, IPG), jnp.int32),
            pltpu.VMEM((GPW, IPG), jnp.int32),
            pltpu.VMEM((RPW,), jnp.int32),
            pltpu.VMEM((RPW,), jnp.int32),
            pltpu.VMEM((2, IPG, D_ITEM), f32),
            pltpu.VMEM((2, IPG, D_CATE), f32),
            pltpu.VMEM((RPW, D_ITEM), f32),
            pltpu.VMEM((RPW, D_CATE), f32),
            pltpu.VMEM((RPW, D_ITEM), f32),
            pltpu.VMEM((RPW, D_ITEM), f32),
            pltpu.SemaphoreType.DMA,
            pltpu.SemaphoreType.DMA,
            pltpu.SemaphoreType.DMA,
            pltpu.SemaphoreType.DMA,
        ],
    )
    return kern(ihs, chs, us, its, Ti, Tc, Tu, Tt)


def _mlp_body(x, w1, b1, w2, b2, w3, b3, out):
    h = jnp.maximum(jnp.dot(x[...], w1[...],
                            preferred_element_type=jnp.float32) + b1[...], 0.0)
    h = jnp.maximum(jnp.dot(h, w2[...],
                            preferred_element_type=jnp.float32) + b2[...], 0.0)
    h = jnp.maximum(jnp.dot(h, w3[...],
                            preferred_element_type=jnp.float32) + b3[...], 0.0)
    out[...] = h


def _mlp(x, w1, b1, w2, b2, w3, b3, blk=512):
    grid = (B // blk,)
    full = lambda shape: pl.BlockSpec(shape, lambda i: (0, 0))
    return pl.pallas_call(
        _mlp_body,
        grid=grid,
        in_specs=[
            pl.BlockSpec((blk, FF_IN), lambda i: (i, 0)),
            full((FF_IN, HID)), full((1, HID)),
            full((HID, HID)), full((1, HID)),
            full((HID, HID)), full((1, HID)),
        ],
        out_specs=pl.BlockSpec((blk, HID), lambda i: (i, 0)),
        out_shape=jax.ShapeDtypeStruct((B, HID), jnp.float32),
    )(x, w1, b1, w2, b2, w3, b3)


def _comb_body(ih, ch, s1, s2, eu, et, h, wc, bc, out):
    f32 = jnp.float32
    cnt1 = jnp.sum((ih[...] != 0).astype(f32), axis=1, keepdims=True)
    cnt2 = jnp.sum((ch[...] != 0).astype(f32), axis=1, keepdims=True)
    a1 = s1[...] / cnt1
    a2 = s2[...] / cnt2
    feats = jnp.concatenate([a1, a2, eu[...], et[...], h[...]], axis=-1)
    out[...] = jnp.dot(feats, wc[...], preferred_element_type=f32) + bc[0, 0]


def _combine(ih, ch, s1, s2, eu, et, h, wc, bc, blk=512):
    grid = (B // blk,)
    row = lambda d: pl.BlockSpec((blk, d), lambda i: (i, 0))
    full = lambda shape: pl.BlockSpec(shape, lambda i: (0, 0))
    return pl.pallas_call(
        _comb_body,
        grid=grid,
        in_specs=[
            row(L), row(L), row(D_ITEM), row(D_CATE), row(D_ITEM), row(D_ITEM),
            row(HID), full((D_ITEM * 3 + D_CATE + HID, 1)), full((1, 1)),
        ],
        out_specs=pl.BlockSpec((blk, 1), lambda i: (i, 0)),
        out_shape=jax.ShapeDtypeStruct((B, 1), jnp.float32),
    )(ih, ch, s1, s2, eu, et, h, wc, bc)


def kernel(item_hist_seq, cate_hist_seq, user_seq, item_seq, s_num_feats,
           T_item_hist, T_cate_hist, T_user, T_item,
           W1, b1, W2, b2, W3, b3, Wc, bc):
    ih = item_hist_seq.astype(jnp.int32)
    ch = cate_hist_seq.astype(jnp.int32)
    us = user_seq.astype(jnp.int32)
    its = item_seq.astype(jnp.int32)
    ihs = ih.reshape(B // GROUP, IPG)
    chs = ch.reshape(B // GROUP, IPG)

    sum1, sum2, eu, et = _sc_pool(ihs, chs, us, its,
                                  T_item_hist, T_cate_hist, T_user, T_item)
    h = _mlp(s_num_feats, W1, b1.reshape(1, HID), W2, b2.reshape(1, HID),
             W3, b3.reshape(1, HID))
    out = _combine(ih, ch, sum1, sum2, eu, et, h, Wc, bc.reshape(1, 1))
    return out
